# baseline (speedup 1.0000x reference)
import functools

import jax
import jax.numpy as jnp
from jax import lax
from jax.experimental import pallas as pl
from jax.experimental.pallas import tpu as pltpu
from jax.experimental.pallas import tpu_sc as plsc

N = 10000
E = 320000
D = 128
G = 64

CE = 64
CPT = 320
NBUF = 4
PHASE = 40
TOT_CHUNKS = 16 * CPT
EP = TOT_CHUNKS * CE
NP = 10112
ZROWS = NP // 16

_mesh = plsc.VectorSubcoreMesh(core_axis_name="c", subcore_axis_name="s")


@functools.partial(
    pl.kernel,
    out_type=jax.ShapeDtypeStruct((NP, D), jnp.float32),
    mesh=_mesh,
    scratch_types=[
        pltpu.VMEM((PHASE, CE), jnp.int32),
        pltpu.VMEM((PHASE, CE), jnp.int32),
        [pltpu.VMEM((CE, D), jnp.float32)] * NBUF,
        pltpu.VMEM_SHARED((NP, D), jnp.float32),
        [pltpu.SemaphoreType.DMA] * NBUF,
    ],
)
def _segsum(h_hbm, srcf_hbm, dst3_hbm, zeros_hbm, out_hbm,
            src_v, dst_v, bufs, acc, sems):
    c = lax.axis_index("c")
    s = lax.axis_index("s")

    def run_phase(base):
        pltpu.sync_copy(srcf_hbm.at[pl.ds(base, PHASE)], src_v)
        pltpu.sync_copy(dst3_hbm.at[pl.ds(base, PHASE)], dst_v)
        for k in range(NBUF):
            pltpu.async_copy(h_hbm.at[src_v.at[k]], bufs[k], sems[k])

        def body(i, carry):
            for k in range(NBUF):
                cidx = NBUF * i + k
                pltpu.make_async_copy(h_hbm.at[pl.ds(0, CE)],
                                      bufs[k], sems[k]).wait()
                pltpu.sync_copy(bufs[k], acc.at[dst_v.at[cidx]], add=True)
                pltpu.async_copy(h_hbm.at[src_v.at[cidx + NBUF]],
                                 bufs[k], sems[k])
            return carry

        lax.fori_loop(0, PHASE // NBUF - 1, body, 0)
        for k in range(NBUF):
            pltpu.make_async_copy(h_hbm.at[pl.ds(0, CE)],
                                  bufs[k], sems[k]).wait()
            pltpu.sync_copy(bufs[k],
                            acc.at[dst_v.at[PHASE - NBUF + k]], add=True)

    @pl.when(c == 0)
    def _():
        pltpu.sync_copy(zeros_hbm, acc.at[pl.ds(s * ZROWS, ZROWS)])
        plsc.subcore_barrier()

        for p in range(CPT // PHASE):
            run_phase(s * CPT + p * PHASE)

        plsc.subcore_barrier()
        pltpu.sync_copy(acc.at[pl.ds(s * ZROWS, ZROWS)],
                        out_hbm.at[pl.ds(s * ZROWS, ZROWS)])


_DOT = functools.partial(jnp.dot, precision=lax.Precision.HIGHEST,
                         preferred_element_type=jnp.float32)


def _conv1_body(h_ref, agg_ref, wa_ref, ba_ref, wb_ref, bb_ref, out_ref):
    h = h_ref[...] + agg_ref[...]
    t = jnp.maximum(_DOT(h, wa_ref[...]) + ba_ref[...], 0.0)
    o = _DOT(t, wb_ref[...]) + bb_ref[...]
    out_ref[...] = jnp.maximum(o, 0.0)


def _conv2_body(h_ref, agg_ref, batch_ref, wa_ref, ba_ref, wb_ref, bb_ref,
                wm1_ref, bm1_ref, wm2_ref, bm2_ref, out_ref,
                sum_ref, cnt_ref, *, blk, nblk):
    i = pl.program_id(0)

    @pl.when(i == 0)
    def _():
        sum_ref[...] = jnp.zeros((G, D), jnp.float32)
        cnt_ref[...] = jnp.zeros((G, D), jnp.float32)

    h = h_ref[...] + agg_ref[...]
    t = jnp.maximum(_DOT(h, wa_ref[...]) + ba_ref[...], 0.0)
    h2 = jnp.maximum(_DOT(t, wb_ref[...]) + bb_ref[...], 0.0)

    b = batch_ref[0, 0, :]
    onehot = (lax.broadcasted_iota(jnp.int32, (G, blk), 0)
              == b[None, :]).astype(jnp.float32)
    sum_ref[...] += _DOT(onehot, h2)
    cnt_ref[...] += _DOT(onehot, jnp.ones((blk, D), jnp.float32))

    @pl.when(i == nblk - 1)
    def _():
        pooled = sum_ref[...] / jnp.maximum(cnt_ref[...], 1.0)
        r = jnp.maximum(_DOT(pooled, wm1_ref[...]) + bm1_ref[...], 0.0)
        out_ref[...] = _DOT(r, wm2_ref[...]) + bm2_ref[...]


_BLK = 1000
_NBLK = N // _BLK

_w_spec = pl.BlockSpec((D, D), lambda i: (0, 0))
_b_spec = pl.BlockSpec((1, D), lambda i: (0, 0))

_conv1 = pl.pallas_call(
    _conv1_body,
    grid=(_NBLK,),
    in_specs=[
        pl.BlockSpec((_BLK, D), lambda i: (i, 0)),
        pl.BlockSpec((_BLK, D), lambda i: (i, 0)),
        _w_spec, _b_spec, _w_spec, _b_spec,
    ],
    out_specs=pl.BlockSpec((_BLK, D), lambda i: (i, 0)),
    out_shape=jax.ShapeDtypeStruct((N, D), jnp.float32),
)

_conv2 = pl.pallas_call(
    functools.partial(_conv2_body, blk=_BLK, nblk=_NBLK),
    grid=(_NBLK,),
    in_specs=[
        pl.BlockSpec((_BLK, D), lambda i: (i, 0)),
        pl.BlockSpec((_BLK, D), lambda i: (i, 0)),
        pl.BlockSpec((1, 1, _BLK), lambda i: (i, 0, 0)),
        _w_spec, _b_spec, _w_spec, _b_spec,
        _w_spec, _b_spec, _w_spec, _b_spec,
    ],
    out_specs=pl.BlockSpec((G, D), lambda i: (0, 0)),
    out_shape=jax.ShapeDtypeStruct((G, D), jnp.float32),
    scratch_shapes=[
        pltpu.VMEM((G, D), jnp.float32),
        pltpu.VMEM((G, D), jnp.float32),
    ],
)


def kernel(x, edge_index, batch, W1a, b1a, W1b, b1b, W2a, b2a, W2b, b2b,
           Wm1, bm1, Wm2, bm2):
    src = edge_index[0]
    dst = edge_index[1]
    pad = EP - E
    srcp = jnp.concatenate([src, jnp.zeros((pad,), jnp.int32)])
    dstp = jnp.concatenate([dst, jnp.full((pad,), N, jnp.int32)])
    srcf = srcp.reshape(TOT_CHUNKS, CE)
    dst3 = dstp.reshape(TOT_CHUNKS, CE)
    zeros = jnp.zeros((ZROWS, D), jnp.float32)
    batch3 = batch.reshape(_NBLK, 1, _BLK)

    b1a_, b1b_ = b1a.reshape(1, D), b1b.reshape(1, D)
    b2a_, b2b_ = b2a.reshape(1, D), b2b.reshape(1, D)
    bm1_, bm2_ = bm1.reshape(1, D), bm2.reshape(1, D)

    agg1 = _segsum(x, srcf, dst3, zeros)
    h1 = _conv1(x, agg1, W1a, b1a_, W1b, b1b_)
    agg2 = _segsum(h1, srcf, dst3, zeros)
    out = _conv2(h1, agg2, batch3, W2a, b2a_, W2b, b2b_,
                 Wm1, bm1_, Wm2, bm2_)
    return out


# speedup vs baseline: 1.3101x; 1.3101x over previous
import functools

import jax
import jax.numpy as jnp
from jax import lax
from jax.experimental import pallas as pl
from jax.experimental.pallas import tpu as pltpu
from jax.experimental.pallas import tpu_sc as plsc

N = 10000
E = 320000
D = 128
G = 64

CE = 128
CH0 = 144
CH1 = 16
NBUF = 2
PHASE = 16
TOT_CHUNKS = 16 * (CH0 + CH1)
EP = TOT_CHUNKS * CE
NP = 10112
ZROWS = NP // 16

_mesh = plsc.VectorSubcoreMesh(core_axis_name="c", subcore_axis_name="s")


@functools.partial(
    pl.kernel,
    out_type=jax.ShapeDtypeStruct((2, NP, D), jnp.float32),
    mesh=_mesh,
    scratch_types=[
        pltpu.VMEM((PHASE, CE), jnp.int32),
        pltpu.VMEM((PHASE, CE), jnp.int32),
        [pltpu.VMEM((CE, D), jnp.float32)] * NBUF,
        pltpu.VMEM_SHARED((NP, D), jnp.float32),
        [pltpu.SemaphoreType.DMA] * NBUF,
    ],
)
def _segsum(h_hbm, srcf_hbm, dst3_hbm, zeros_hbm, out_hbm,
            src_v, dst_v, bufs, acc, sems):
    c = lax.axis_index("c")
    s = lax.axis_index("s")

    def run_phase(base):
        pltpu.sync_copy(srcf_hbm.at[pl.ds(base, PHASE)], src_v)
        pltpu.sync_copy(dst3_hbm.at[pl.ds(base, PHASE)], dst_v)
        for k in range(NBUF):
            pltpu.async_copy(h_hbm.at[src_v.at[k]], bufs[k], sems[k])

        def body(i, carry):
            for k in range(NBUF):
                cidx = NBUF * i + k
                pltpu.make_async_copy(h_hbm.at[pl.ds(0, CE)],
                                      bufs[k], sems[k]).wait()
                pltpu.sync_copy(bufs[k], acc.at[dst_v.at[cidx]], add=True)
                pltpu.async_copy(h_hbm.at[src_v.at[cidx + NBUF]],
                                 bufs[k], sems[k])
            return carry

        lax.fori_loop(0, PHASE // NBUF - 1, body, 0)
        for k in range(NBUF):
            pltpu.make_async_copy(h_hbm.at[pl.ds(0, CE)],
                                  bufs[k], sems[k]).wait()
            pltpu.sync_copy(bufs[k],
                            acc.at[dst_v.at[PHASE - NBUF + k]], add=True)

    pltpu.sync_copy(zeros_hbm, acc.at[pl.ds(s * ZROWS, ZROWS)])
    plsc.subcore_barrier()

    @pl.when(c == 0)
    def _():
        for p in range(CH0 // PHASE):
            run_phase(s * CH0 + p * PHASE)

    @pl.when(c == 1)
    def _():
        for p in range(CH1 // PHASE):
            run_phase(16 * CH0 + s * CH1 + p * PHASE)

    plsc.subcore_barrier()
    pltpu.sync_copy(acc.at[pl.ds(s * ZROWS, ZROWS)],
                    out_hbm.at[c, pl.ds(s * ZROWS, ZROWS)])


_DOT = functools.partial(jnp.dot, precision=lax.Precision.HIGHEST,
                         preferred_element_type=jnp.float32)


def _conv1_body(h_ref, agg_ref, wa_ref, ba_ref, wb_ref, bb_ref, out_ref):
    h = h_ref[...] + agg_ref[0] + agg_ref[1]
    t = jnp.maximum(_DOT(h, wa_ref[...]) + ba_ref[...], 0.0)
    o = _DOT(t, wb_ref[...]) + bb_ref[...]
    out_ref[...] = jnp.maximum(o, 0.0)


def _conv2_body(h_ref, agg_ref, batch_ref, wa_ref, ba_ref, wb_ref, bb_ref,
                wm1_ref, bm1_ref, wm2_ref, bm2_ref, out_ref,
                sum_ref, cnt_ref, *, blk, nblk):
    i = pl.program_id(0)

    @pl.when(i == 0)
    def _():
        sum_ref[...] = jnp.zeros((G, D), jnp.float32)
        cnt_ref[...] = jnp.zeros((G, D), jnp.float32)

    h = h_ref[...] + agg_ref[0] + agg_ref[1]
    t = jnp.maximum(_DOT(h, wa_ref[...]) + ba_ref[...], 0.0)
    h2 = jnp.maximum(_DOT(t, wb_ref[...]) + bb_ref[...], 0.0)

    b = batch_ref[0, 0, :]
    onehot = (lax.broadcasted_iota(jnp.int32, (G, blk), 0)
              == b[None, :]).astype(jnp.float32)
    sum_ref[...] += _DOT(onehot, h2)
    cnt_ref[...] += _DOT(onehot, jnp.ones((blk, D), jnp.float32))

    @pl.when(i == nblk - 1)
    def _():
        pooled = sum_ref[...] / jnp.maximum(cnt_ref[...], 1.0)
        r = jnp.maximum(_DOT(pooled, wm1_ref[...]) + bm1_ref[...], 0.0)
        out_ref[...] = _DOT(r, wm2_ref[...]) + bm2_ref[...]


_BLK = 1000
_NBLK = N // _BLK

_w_spec = pl.BlockSpec((D, D), lambda i: (0, 0))
_b_spec = pl.BlockSpec((1, D), lambda i: (0, 0))

_conv1 = pl.pallas_call(
    _conv1_body,
    grid=(_NBLK,),
    in_specs=[
        pl.BlockSpec((_BLK, D), lambda i: (i, 0)),
        pl.BlockSpec((2, _BLK, D), lambda i: (0, i, 0)),
        _w_spec, _b_spec, _w_spec, _b_spec,
    ],
    out_specs=pl.BlockSpec((_BLK, D), lambda i: (i, 0)),
    out_shape=jax.ShapeDtypeStruct((N, D), jnp.float32),
)

_conv2 = pl.pallas_call(
    functools.partial(_conv2_body, blk=_BLK, nblk=_NBLK),
    grid=(_NBLK,),
    in_specs=[
        pl.BlockSpec((_BLK, D), lambda i: (i, 0)),
        pl.BlockSpec((2, _BLK, D), lambda i: (0, i, 0)),
        pl.BlockSpec((1, 1, _BLK), lambda i: (i, 0, 0)),
        _w_spec, _b_spec, _w_spec, _b_spec,
        _w_spec, _b_spec, _w_spec, _b_spec,
    ],
    out_specs=pl.BlockSpec((G, D), lambda i: (0, 0)),
    out_shape=jax.ShapeDtypeStruct((G, D), jnp.float32),
    scratch_shapes=[
        pltpu.VMEM((G, D), jnp.float32),
        pltpu.VMEM((G, D), jnp.float32),
    ],
)


def kernel(x, edge_index, batch, W1a, b1a, W1b, b1b, W2a, b2a, W2b, b2b,
           Wm1, bm1, Wm2, bm2):
    src = edge_index[0]
    dst = edge_index[1]
    pad = EP - E
    srcp = jnp.concatenate([src, jnp.zeros((pad,), jnp.int32)])
    dstp = jnp.concatenate([dst, jnp.full((pad,), N, jnp.int32)])
    srcf = srcp.reshape(TOT_CHUNKS, CE)
    dst3 = dstp.reshape(TOT_CHUNKS, CE)
    zeros = jnp.zeros((ZROWS, D), jnp.float32)
    batch3 = batch.reshape(_NBLK, 1, _BLK)

    b1a_, b1b_ = b1a.reshape(1, D), b1b.reshape(1, D)
    b2a_, b2b_ = b2a.reshape(1, D), b2b.reshape(1, D)
    bm1_, bm2_ = bm1.reshape(1, D), bm2.reshape(1, D)

    agg1 = _segsum(x, srcf, dst3, zeros)
    h1 = _conv1(x, agg1, W1a, b1a_, W1b, b1b_)
    agg2 = _segsum(h1, srcf, dst3, zeros)
    out = _conv2(h1, agg2, batch3, W2a, b2a_, W2b, b2b_,
                 Wm1, bm1_, Wm2, bm2_)
    return out


# speedup vs baseline: 1.3280x; 1.0136x over previous
import functools

import jax
import jax.numpy as jnp
from jax import lax
from jax.experimental import pallas as pl
from jax.experimental.pallas import tpu as pltpu
from jax.experimental.pallas import tpu_sc as plsc

N = 10000
E = 320000
D = 128
G = 64

CE = 128
CH0 = 152
CH1 = 8
NBUF = 2
PHASE = 40
PHASES0 = (40, 40, 40, 32)
PHASES1 = (8,)
TOT_CHUNKS = 16 * (CH0 + CH1)
EP = TOT_CHUNKS * CE
NP = 10112
ZROWS = NP // 16

_mesh = plsc.VectorSubcoreMesh(core_axis_name="c", subcore_axis_name="s")


@functools.partial(
    pl.kernel,
    out_type=jax.ShapeDtypeStruct((2, NP, D), jnp.float32),
    mesh=_mesh,
    scratch_types=[
        pltpu.VMEM((PHASE, CE), jnp.int32),
        pltpu.VMEM((PHASE, CE), jnp.int32),
        [pltpu.VMEM((CE, D), jnp.float32)] * NBUF,
        pltpu.VMEM_SHARED((NP, D), jnp.float32),
        [pltpu.SemaphoreType.DMA] * NBUF,
    ],
)
def _segsum(h_hbm, srcf_hbm, dst3_hbm, zeros_hbm, out_hbm,
            src_v, dst_v, bufs, acc, sems):
    c = lax.axis_index("c")
    s = lax.axis_index("s")

    def run_phase(base, n):
        pltpu.sync_copy(srcf_hbm.at[pl.ds(base, n)], src_v.at[pl.ds(0, n)])
        pltpu.sync_copy(dst3_hbm.at[pl.ds(base, n)], dst_v.at[pl.ds(0, n)])
        for k in range(NBUF):
            pltpu.async_copy(h_hbm.at[src_v.at[k]], bufs[k], sems[k])

        def body(i, carry):
            for k in range(NBUF):
                cidx = NBUF * i + k
                pltpu.make_async_copy(h_hbm.at[pl.ds(0, CE)],
                                      bufs[k], sems[k]).wait()
                pltpu.sync_copy(bufs[k], acc.at[dst_v.at[cidx]], add=True)
                pltpu.async_copy(h_hbm.at[src_v.at[cidx + NBUF]],
                                 bufs[k], sems[k])
            return carry

        lax.fori_loop(0, n // NBUF - 1, body, 0)
        for k in range(NBUF):
            pltpu.make_async_copy(h_hbm.at[pl.ds(0, CE)],
                                  bufs[k], sems[k]).wait()
            pltpu.sync_copy(bufs[k],
                            acc.at[dst_v.at[n - NBUF + k]], add=True)

    pltpu.sync_copy(zeros_hbm, acc.at[pl.ds(s * ZROWS, ZROWS)])
    plsc.subcore_barrier()

    @pl.when(c == 0)
    def _():
        off = 0
        for n in PHASES0:
            run_phase(s * CH0 + off, n)
            off += n

    @pl.when(c == 1)
    def _():
        off = 0
        for n in PHASES1:
            run_phase(16 * CH0 + s * CH1 + off, n)
            off += n

    plsc.subcore_barrier()
    pltpu.sync_copy(acc.at[pl.ds(s * ZROWS, ZROWS)],
                    out_hbm.at[c, pl.ds(s * ZROWS, ZROWS)])


_DOT = functools.partial(jnp.dot, precision=lax.Precision.HIGHEST,
                         preferred_element_type=jnp.float32)


def _conv1_body(h_ref, agg_ref, wa_ref, ba_ref, wb_ref, bb_ref, out_ref):
    h = h_ref[...] + agg_ref[0] + agg_ref[1]
    t = jnp.maximum(_DOT(h, wa_ref[...]) + ba_ref[...], 0.0)
    o = _DOT(t, wb_ref[...]) + bb_ref[...]
    out_ref[...] = jnp.maximum(o, 0.0)


def _conv2_body(h_ref, agg_ref, batch_ref, wa_ref, ba_ref, wb_ref, bb_ref,
                wm1_ref, bm1_ref, wm2_ref, bm2_ref, out_ref,
                sum_ref, cnt_ref, *, blk, nblk):
    i = pl.program_id(0)

    @pl.when(i == 0)
    def _():
        sum_ref[...] = jnp.zeros((G, D), jnp.float32)
        cnt_ref[...] = jnp.zeros((G, D), jnp.float32)

    h = h_ref[...] + agg_ref[0] + agg_ref[1]
    t = jnp.maximum(_DOT(h, wa_ref[...]) + ba_ref[...], 0.0)
    h2 = jnp.maximum(_DOT(t, wb_ref[...]) + bb_ref[...], 0.0)

    b = batch_ref[0, 0, :]
    onehot = (lax.broadcasted_iota(jnp.int32, (G, blk), 0)
              == b[None, :]).astype(jnp.float32)
    sum_ref[...] += _DOT(onehot, h2)
    cnt_ref[...] += _DOT(onehot, jnp.ones((blk, D), jnp.float32))

    @pl.when(i == nblk - 1)
    def _():
        pooled = sum_ref[...] / jnp.maximum(cnt_ref[...], 1.0)
        r = jnp.maximum(_DOT(pooled, wm1_ref[...]) + bm1_ref[...], 0.0)
        out_ref[...] = _DOT(r, wm2_ref[...]) + bm2_ref[...]


_BLK = 1000
_NBLK = N // _BLK

_w_spec = pl.BlockSpec((D, D), lambda i: (0, 0))
_b_spec = pl.BlockSpec((1, D), lambda i: (0, 0))

_conv1 = pl.pallas_call(
    _conv1_body,
    grid=(_NBLK,),
    in_specs=[
        pl.BlockSpec((_BLK, D), lambda i: (i, 0)),
        pl.BlockSpec((2, _BLK, D), lambda i: (0, i, 0)),
        _w_spec, _b_spec, _w_spec, _b_spec,
    ],
    out_specs=pl.BlockSpec((_BLK, D), lambda i: (i, 0)),
    out_shape=jax.ShapeDtypeStruct((N, D), jnp.float32),
)

_conv2 = pl.pallas_call(
    functools.partial(_conv2_body, blk=_BLK, nblk=_NBLK),
    grid=(_NBLK,),
    in_specs=[
        pl.BlockSpec((_BLK, D), lambda i: (i, 0)),
        pl.BlockSpec((2, _BLK, D), lambda i: (0, i, 0)),
        pl.BlockSpec((1, 1, _BLK), lambda i: (i, 0, 0)),
        _w_spec, _b_spec, _w_spec, _b_spec,
        _w_spec, _b_spec, _w_spec, _b_spec,
    ],
    out_specs=pl.BlockSpec((G, D), lambda i: (0, 0)),
    out_shape=jax.ShapeDtypeStruct((G, D), jnp.float32),
    scratch_shapes=[
        pltpu.VMEM((G, D), jnp.float32),
        pltpu.VMEM((G, D), jnp.float32),
    ],
)


def kernel(x, edge_index, batch, W1a, b1a, W1b, b1b, W2a, b2a, W2b, b2b,
           Wm1, bm1, Wm2, bm2):
    src = edge_index[0]
    dst = edge_index[1]
    pad = EP - E
    srcp = jnp.concatenate([src, jnp.zeros((pad,), jnp.int32)])
    dstp = jnp.concatenate([dst, jnp.full((pad,), N, jnp.int32)])
    srcf = srcp.reshape(TOT_CHUNKS, CE)
    dst3 = dstp.reshape(TOT_CHUNKS, CE)
    zeros = jnp.zeros((ZROWS, D), jnp.float32)
    batch3 = batch.reshape(_NBLK, 1, _BLK)

    b1a_, b1b_ = b1a.reshape(1, D), b1b.reshape(1, D)
    b2a_, b2b_ = b2a.reshape(1, D), b2b.reshape(1, D)
    bm1_, bm2_ = bm1.reshape(1, D), bm2.reshape(1, D)

    agg1 = _segsum(x, srcf, dst3, zeros)
    h1 = _conv1(x, agg1, W1a, b1a_, W1b, b1b_)
    agg2 = _segsum(h1, srcf, dst3, zeros)
    out = _conv2(h1, agg2, batch3, W2a, b2a_, W2b, b2b_,
                 Wm1, bm1_, Wm2, bm2_)
    return out
